# baseline (speedup 1.0000x reference)
import functools

import jax
import jax.numpy as jnp
from jax import lax
from jax.experimental import pallas as pl
from jax.experimental.pallas import tpu as pltpu
from jax.experimental.pallas import tpu_sc as plsc

_EM = 128
_HID = [400, 400, 400]
_MAIN_IN = 39
_MAIN_OUT = 8
_TASKS = 50
_GEN_HID = 256
_DIMS = [_MAIN_IN] + _HID + [_MAIN_OUT]
_NS = [_DIMS[i] * _DIMS[i + 1] for i in range(len(_DIMS) - 1)]
_OFFS = [0, 15600, 175600, 335600]
_TOTAL = sum(_NS)

_C = 8192
_RELMAX = [max(0, n - _C) for n in _NS]
_CAP = 8192
_NCH = [-(-n // _C) for n in _NS]

_BLK = 2048


def _make_noise():
    import numpy as np

    def build():
        key = jax.random.key(42)
        parts = []
        for i, n in enumerate(_NS):
            e = jax.random.exponential(jax.random.fold_in(key, i), (_TASKS, n),
                                       dtype=jnp.float32)
            parts.append(jnp.log(e))
        return jnp.concatenate(parts, axis=1)

    cpu = jax.devices("cpu")[0]
    with jax.default_device(cpu):
        return np.asarray(build())


_NOISE = _make_noise()


def _h_body(emb_ref, w1_ref, b1_ref, h_ref):
    acc = jnp.dot(emb_ref[...], w1_ref[...], preferred_element_type=jnp.float32)
    h_ref[...] = jnp.maximum(acc + b1_ref[...], 0.0)


def _g_body(h_ref, w2_ref, b2_ref, nz_ref, g_ref):
    acc = jnp.dot(h_ref[...], w2_ref[...], preferred_element_type=jnp.float32)
    g_ref[...] = acc + b2_ref[...] - nz_ref[...]


def _compute_g(emb, W1, b1, W2, b2):
    h = pl.pallas_call(
        _h_body,
        out_shape=jax.ShapeDtypeStruct((_TASKS, _GEN_HID), jnp.float32),
    )(emb, W1, b1.reshape(1, _GEN_HID))
    nblk = -(-_TOTAL // _BLK)
    g = pl.pallas_call(
        _g_body,
        grid=(nblk,),
        in_specs=[
            pl.BlockSpec((_TASKS, _GEN_HID), lambda j: (0, 0)),
            pl.BlockSpec((_GEN_HID, _BLK), lambda j: (0, j)),
            pl.BlockSpec((1, _BLK), lambda j: (0, j)),
            pl.BlockSpec((_TASKS, _BLK), lambda j: (0, j)),
        ],
        out_specs=pl.BlockSpec((_TASKS, _BLK), lambda j: (0, j)),
        out_shape=jax.ShapeDtypeStruct((_TASKS, _TOTAL), jnp.float32),
    )(h, W2, b2.reshape(1, _TOTAL), _NOISE)
    return g


def _lane():
    return lax.iota(jnp.int32, 16)


def _to_key(v):
    u = lax.bitcast_convert_type(v, jnp.int32)
    return u ^ (lax.shift_right_arithmetic(u, 31) & jnp.int32(0x7FFFFFFF))


def _scan_top(hist_ref, ngroups, k_rem, nsub=4):
    lane = _lane()

    def body(i, carry):
        above, found, b, kout, cbin = carry
        j = ngroups - 1 - i
        v = hist_ref[pl.ds(j * 16, 16)]
        hist_ref[pl.ds(j * 16, 16)] = jnp.zeros((16,), jnp.int32)
        for s in range(1, nsub):
            v = v + hist_ref[pl.ds(s * 4096 + j * 16, 16)]
            hist_ref[pl.ds(s * 4096 + j * 16, 16)] = jnp.zeros((16,), jnp.int32)
        gsum = jnp.sum(v)
        rev = lax.rev(v, (0,))
        c = plsc.cumsum(rev)
        hit = jnp.logical_and(found == 0, above + gsum >= k_rem)
        crossed = (above + c) >= k_rem
        L = jnp.min(jnp.where(crossed, lane, 16))
        cL = jnp.max(jnp.where(lane == L, c, 0))
        rL = jnp.max(jnp.where(lane == L, rev, 0))
        newb = j * 16 + 15 - L
        newk = k_rem - (above + cL - rL)
        b = jnp.where(hit, newb, b)
        kout = jnp.where(hit, newk, kout)
        cbin = jnp.where(hit, rL, cbin)
        found = jnp.where(hit, 1, found)
        return above + gsum, found, b, kout, cbin

    _, _, b, kout, cbin = lax.fori_loop(
        0, ngroups, body, (jnp.int32(0), jnp.int32(0), jnp.int32(0),
                           jnp.int32(k_rem), jnp.int32(0)))
    return b, kout, cbin


def _rel_of(ci, relmax):
    return jnp.minimum(ci * _C, relmax)


def _dma_start(g_ref, gbase, relmax, ci, buf, sem):
    pltpu.async_copy(g_ref.at[pl.ds(gbase + _rel_of(ci, relmax), _C)], buf,
                     sem)


def _dma_wait(g_ref, buf, sem):
    pltpu.make_async_copy(g_ref.at[pl.ds(0, _C)], buf, sem).wait()


def _dbuf_loop(g_ref, bufs, sems, gbase, n, relmax, nch, proc, init):
    b0, b1 = bufs
    s0, s1 = sems
    npair = (nch + 1) // 2
    _dma_start(g_ref, gbase, relmax, 0, b0, s0)

    def ovl_of(ci):
        return jnp.where(ci < nch, ci * _C - _rel_of(ci, relmax), _C)

    def lim_of(ci):
        return jnp.where(ci < nch, n - _rel_of(ci, relmax), 0)

    def pair(p, carry):
        ci0 = 2 * p
        ci1 = ci0 + 1
        _dma_start(g_ref, gbase, relmax, ci1, b1, s1)
        _dma_wait(g_ref, b0, s0)
        carry = proc(b0, ci0, ovl_of(ci0), lim_of(ci0), carry)

        @pl.when(ci0 + 2 < 2 * npair)
        def _():
            _dma_start(g_ref, gbase, relmax, ci0 + 2, b0, s0)
        _dma_wait(g_ref, b1, s1)
        carry = proc(b1, ci1, ovl_of(ci1), lim_of(ci1), carry)
        return carry

    return lax.fori_loop(0, npair, pair, init)


def _sel_body(g_ref, out0, out1, out2, out3, dbuf0, dbuf1, obuf, hist, cand,
              sem0, sem1):
    lane = _lane()
    wid = lax.axis_index("s") * 2 + lax.axis_index("c")

    def z(i, _):
        hist[pl.ds(i * 16, 16)] = jnp.zeros((16,), jnp.int32)
        return 0
    lax.fori_loop(0, 1024, z, 0)

    def run_task(t):
        l = t // _TASKS
        r = t % _TASKS
        n = jnp.where(l == 0, _NS[0],
                      jnp.where(l == 3, _NS[3], _NS[1])).astype(jnp.int32)
        off = jnp.where(l == 0, _OFFS[0],
                        jnp.where(l == 1, _OFFS[1],
                                  jnp.where(l == 2, _OFFS[2], _OFFS[3])))
        nch = jnp.where(l == 0, _NCH[0],
                        jnp.where(l == 3, _NCH[3], _NCH[1]))
        relmax = jnp.where(l == 0, _RELMAX[0],
                           jnp.where(l == 3, _RELMAX[3], _RELMAX[1]))
        k = n // 2
        gbase = r * _TOTAL + off

        def hist_proc(buf, ci, ovl, lim, carry):
            @plsc.parallel_loop(0, _C // 16, unroll=4)
            def grp(gi):
                key = _to_key(buf[pl.ds(gi * 16, 16)])
                idx = gi * 16 + lane
                valid = jnp.logical_and(idx >= ovl, idx < lim)
                d = jnp.where(valid,
                              lax.shift_right_arithmetic(key, 20) + 2048, 0)
                d = d | lax.shift_left(gi & 3, 12)
                cnt, last = plsc.scan_count(d)
                plsc.addupdate_scatter(hist, [d], cnt, mask=last)
            return carry
        _dbuf_loop(g_ref, (dbuf0, dbuf1), (sem0, sem1), gbase, n, relmax,
                   nch, hist_proc, 0)

        b1, k1, _ = _scan_top(hist, 256, k)

        def coll_proc(buf, ci, ovl, lim, cnt_sofar):
            def grp(gi, cnt2):
                key = _to_key(buf[pl.ds(gi * 16, 16)])
                idx = gi * 16 + lane
                valid = jnp.logical_and(idx >= ovl, idx < lim)
                d = lax.shift_right_arithmetic(key, 20) + 2048
                sel = jnp.logical_and(valid, d == b1)
                sel = jnp.logical_and(sel, lane + cnt2 < _CAP)
                pos = jnp.minimum(cnt2, _CAP - 16)
                plsc.store_compressed(cand.at[pl.ds(pos, 16)], key, mask=sel)
                npick = plsc.all_reduce_population_count(sel)[0]
                return cnt2 + npick
            return lax.fori_loop(0, _C // 16, grp, cnt_sofar)
        cnt1 = _dbuf_loop(g_ref, (dbuf0, dbuf1), (sem0, sem1), gbase, n,
                          relmax, nch, coll_proc, jnp.int32(0))

        ng = (cnt1 + 15) // 16

        @plsc.parallel_loop(0, ng, unroll=4)
        def bh2(gi):
            kv = cand[pl.ds(gi * 16, 16)]
            elig = (gi * 16 + lane) < cnt1
            d = jnp.where(elig,
                          lax.shift_right_arithmetic(kv, 8) & 0xFFF, 0)
            d = d | lax.shift_left(gi & 3, 12)
            cnt, last = plsc.scan_count(d)
            plsc.addupdate_scatter(hist, [d], cnt, mask=last)
        b2l, k2, _ = _scan_top(hist, 256, k1)

        @plsc.parallel_loop(0, ng, unroll=4)
        def bh3(gi):
            kv = cand[pl.ds(gi * 16, 16)]
            elig = jnp.logical_and(
                (gi * 16 + lane) < cnt1,
                (lax.shift_right_arithmetic(kv, 8) & 0xFFF) == b2l)
            d = jnp.where(elig, kv & 0xFF, 0)
            d = d | lax.shift_left(gi & 3, 12)
            cnt, last = plsc.scan_count(d)
            plsc.addupdate_scatter(hist, [d], cnt, mask=last)
        b3, _, _ = _scan_top(hist, 16, k2)

        kstar = (lax.shift_left(b1 - 2048, 20) | lax.shift_left(b2l, 8) | b3)

        def write_proc(buf, ci, ovl, lim, carry):
            @plsc.parallel_loop(0, _C // 16, unroll=4)
            def grp(gi):
                key = _to_key(buf[pl.ds(gi * 16, 16)])
                obuf[pl.ds(gi * 16, 16)] = jnp.where(
                    key >= kstar, jnp.float32(1.0), jnp.float32(0.0))

            rel = _rel_of(ci, relmax)
            obase = r * n + rel
            real = ci < nch

            @pl.when(jnp.logical_and(real, l == 0))
            def _():
                pltpu.sync_copy(obuf, out0.at[pl.ds(obase, _C)])

            @pl.when(jnp.logical_and(real, l == 1))
            def _():
                pltpu.sync_copy(obuf, out1.at[pl.ds(obase, _C)])

            @pl.when(jnp.logical_and(real, l == 2))
            def _():
                pltpu.sync_copy(obuf, out2.at[pl.ds(obase, _C)])

            @pl.when(jnp.logical_and(real, l == 3))
            def _():
                pltpu.sync_copy(obuf.at[pl.ds(0, _NS[3])],
                                out3.at[pl.ds(obase, _NS[3])])
            return carry
        _dbuf_loop(g_ref, (dbuf0, dbuf1), (sem0, sem1), gbase, n, relmax,
                   nch, write_proc, 0)

    def slot_body(slot, _):
        t = wid + slot * 32

        @pl.when(t < 4 * _TASKS)
        def _():
            run_task(t)
        return 0
    lax.fori_loop(0, 7, slot_body, 0)


@functools.partial(jax.jit, static_argnums=())
def _select_masks(gflat):
    gflat = jnp.concatenate([gflat, jnp.zeros((_C,), jnp.float32)])
    f = pl.kernel(
        _sel_body,
        mesh=plsc.VectorSubcoreMesh(core_axis_name="c", subcore_axis_name="s"),
        out_type=[jax.ShapeDtypeStruct((_TASKS * n,), jnp.float32)
                  for n in _NS],
        scratch_types=[
            pltpu.VMEM((_C,), jnp.float32),
            pltpu.VMEM((_C,), jnp.float32),
            pltpu.VMEM((_C,), jnp.float32),
            pltpu.VMEM((4 * 4096,), jnp.int32),
            pltpu.VMEM((_CAP,), jnp.int32),
            pltpu.SemaphoreType.DMA,
            pltpu.SemaphoreType.DMA,
        ],
        compiler_params=pltpu.CompilerParams(needs_layout_passes=False),
    )
    return f(gflat)


def kernel(x, embedding_input, W1, b1, W2, b2):
    g = _compute_g(embedding_input, W1, b1, W2, b2)
    m0, m1, m2, m3 = _select_masks(g.reshape(-1))
    outs = []
    for l, m in enumerate((m0, m1, m2, m3)):
        in_d, out_d = _DIMS[l], _DIMS[l + 1]
        outs.append(m.reshape(_TASKS, out_d, in_d))
    return tuple(outs)


# speedup vs baseline: 1.0242x; 1.0242x over previous
import functools

import jax
import jax.numpy as jnp
from jax import lax
from jax.experimental import pallas as pl
from jax.experimental.pallas import tpu as pltpu
from jax.experimental.pallas import tpu_sc as plsc

_EM = 128
_HID = [400, 400, 400]
_MAIN_IN = 39
_MAIN_OUT = 8
_TASKS = 50
_GEN_HID = 256
_DIMS = [_MAIN_IN] + _HID + [_MAIN_OUT]
_NS = [_DIMS[i] * _DIMS[i + 1] for i in range(len(_DIMS) - 1)]
_OFFS = [0, 15600, 175600, 335600]
_TOTAL = sum(_NS)

_C = 8192
_RELMAX = [max(0, n - _C) for n in _NS]
_CAP = 8192
_NCH = [-(-n // _C) for n in _NS]

_BLK = 2048


def _make_noise():
    import numpy as np

    def build():
        key = jax.random.key(42)
        parts = []
        for i, n in enumerate(_NS):
            e = jax.random.exponential(jax.random.fold_in(key, i), (_TASKS, n),
                                       dtype=jnp.float32)
            parts.append(jnp.log(e))
        return jnp.concatenate(parts, axis=1)

    cpu = jax.devices("cpu")[0]
    with jax.default_device(cpu):
        return np.asarray(build())


_NOISE = _make_noise()


def _h_body(emb_ref, w1_ref, b1_ref, h_ref):
    acc = jnp.dot(emb_ref[...], w1_ref[...], preferred_element_type=jnp.float32)
    h_ref[...] = jnp.maximum(acc + b1_ref[...], 0.0)


def _g_body(h_ref, w2_ref, b2_ref, nz_ref, g_ref):
    acc = jnp.dot(h_ref[...], w2_ref[...], preferred_element_type=jnp.float32)
    g_ref[...] = acc + b2_ref[...] - nz_ref[...]


def _compute_g(emb, W1, b1, W2, b2):
    h = pl.pallas_call(
        _h_body,
        out_shape=jax.ShapeDtypeStruct((_TASKS, _GEN_HID), jnp.float32),
    )(emb, W1, b1.reshape(1, _GEN_HID))
    nblk = -(-_TOTAL // _BLK)
    g = pl.pallas_call(
        _g_body,
        grid=(nblk,),
        in_specs=[
            pl.BlockSpec((_TASKS, _GEN_HID), lambda j: (0, 0)),
            pl.BlockSpec((_GEN_HID, _BLK), lambda j: (0, j)),
            pl.BlockSpec((1, _BLK), lambda j: (0, j)),
            pl.BlockSpec((_TASKS, _BLK), lambda j: (0, j)),
        ],
        out_specs=pl.BlockSpec((_TASKS, _BLK), lambda j: (0, j)),
        out_shape=jax.ShapeDtypeStruct((_TASKS, _TOTAL), jnp.float32),
    )(h, W2, b2.reshape(1, _TOTAL), _NOISE)
    return g


def _lane():
    return lax.iota(jnp.int32, 16)


def _to_key(v):
    u = lax.bitcast_convert_type(v, jnp.int32)
    return u ^ (lax.shift_right_arithmetic(u, 31) & jnp.int32(0x7FFFFFFF))


def _scan_top(hist_ref, ngroups, k_rem, nsub=4):
    lane = _lane()

    def body(i, carry):
        above, found, b, kout, cbin = carry
        j = ngroups - 1 - i
        v = hist_ref[pl.ds(j * 16, 16)]
        hist_ref[pl.ds(j * 16, 16)] = jnp.zeros((16,), jnp.int32)
        for s in range(1, nsub):
            v = v + hist_ref[pl.ds(s * 4096 + j * 16, 16)]
            hist_ref[pl.ds(s * 4096 + j * 16, 16)] = jnp.zeros((16,), jnp.int32)
        gsum = jnp.sum(v)
        rev = lax.rev(v, (0,))
        c = plsc.cumsum(rev)
        hit = jnp.logical_and(found == 0, above + gsum >= k_rem)
        crossed = (above + c) >= k_rem
        L = jnp.min(jnp.where(crossed, lane, 16))
        cL = jnp.max(jnp.where(lane == L, c, 0))
        rL = jnp.max(jnp.where(lane == L, rev, 0))
        newb = j * 16 + 15 - L
        newk = k_rem - (above + cL - rL)
        b = jnp.where(hit, newb, b)
        kout = jnp.where(hit, newk, kout)
        cbin = jnp.where(hit, rL, cbin)
        found = jnp.where(hit, 1, found)
        return above + gsum, found, b, kout, cbin

    _, _, b, kout, cbin = lax.fori_loop(
        0, ngroups, body, (jnp.int32(0), jnp.int32(0), jnp.int32(0),
                           jnp.int32(k_rem), jnp.int32(0)))
    return b, kout, cbin


def _rel_of(ci, relmax):
    return jnp.minimum(ci * _C, relmax)


def _dma_start(g_ref, gbase, relmax, ci, buf, sem):
    pltpu.async_copy(g_ref.at[pl.ds(gbase + _rel_of(ci, relmax), _C)], buf,
                     sem)


def _dma_wait(g_ref, buf, sem):
    pltpu.make_async_copy(g_ref.at[pl.ds(0, _C)], buf, sem).wait()


def _dbuf_loop(g_ref, bufs, sems, gbase, n, relmax, nch, proc, init):
    b0, b1 = bufs
    s0, s1 = sems
    npair = (nch + 1) // 2
    _dma_start(g_ref, gbase, relmax, 0, b0, s0)

    def ovl_of(ci):
        return jnp.where(ci < nch, ci * _C - _rel_of(ci, relmax), _C)

    def lim_of(ci):
        return jnp.where(ci < nch, n - _rel_of(ci, relmax), 0)

    def pair(p, carry):
        ci0 = 2 * p
        ci1 = ci0 + 1
        _dma_start(g_ref, gbase, relmax, ci1, b1, s1)
        _dma_wait(g_ref, b0, s0)
        carry = proc(b0, ci0, ovl_of(ci0), lim_of(ci0), carry)

        @pl.when(ci0 + 2 < 2 * npair)
        def _():
            _dma_start(g_ref, gbase, relmax, ci0 + 2, b0, s0)
        _dma_wait(g_ref, b1, s1)
        carry = proc(b1, ci1, ovl_of(ci1), lim_of(ci1), carry)
        return carry

    return lax.fori_loop(0, npair, pair, init)


def _sel_body(g_ref, out0, out1, out2, out3, dbuf0, dbuf1, obuf, hist, cand,
              pcv, sem0, sem1):
    lane = _lane()
    wid = lax.axis_index("s") * 2 + lax.axis_index("c")

    def z(i, _):
        hist[pl.ds(i * 16, 16)] = jnp.zeros((16,), jnp.int32)
        return 0
    lax.fori_loop(0, 1024, z, 0)

    def run_task(t):
        l = t // _TASKS
        r = t % _TASKS
        n = jnp.where(l == 0, _NS[0],
                      jnp.where(l == 3, _NS[3], _NS[1])).astype(jnp.int32)
        off = jnp.where(l == 0, _OFFS[0],
                        jnp.where(l == 1, _OFFS[1],
                                  jnp.where(l == 2, _OFFS[2], _OFFS[3])))
        nch = jnp.where(l == 0, _NCH[0],
                        jnp.where(l == 3, _NCH[3], _NCH[1]))
        relmax = jnp.where(l == 0, _RELMAX[0],
                           jnp.where(l == 3, _RELMAX[3], _RELMAX[1]))
        k = n // 2
        gbase = r * _TOTAL + off

        def hist_proc(buf, ci, ovl, lim, carry):
            @plsc.parallel_loop(0, _C // 16, unroll=4)
            def grp(gi):
                key = _to_key(buf[pl.ds(gi * 16, 16)])
                idx = gi * 16 + lane
                valid = jnp.logical_and(idx >= ovl, idx < lim)
                d = jnp.where(valid,
                              lax.shift_right_arithmetic(key, 20) + 2048, 0)
                d = d | lax.shift_left(gi & 3, 12)
                cnt, last = plsc.scan_count(d)
                plsc.addupdate_scatter(hist, [d], cnt, mask=last)
            return carry
        _dbuf_loop(g_ref, (dbuf0, dbuf1), (sem0, sem1), gbase, n, relmax,
                   nch, hist_proc, 0)

        b1, k1, _ = _scan_top(hist, 256, k)

        def coll_proc(buf, ci, ovl, lim, cnt_sofar):
            def selgrp(gi):
                key = _to_key(buf[pl.ds(gi * 16, 16)])
                idx = gi * 16 + lane
                valid = jnp.logical_and(idx >= ovl, idx < lim)
                d = lax.shift_right_arithmetic(key, 20) + 2048
                return key, jnp.logical_and(valid, d == b1)

            @plsc.parallel_loop(0, _C // 16, unroll=4)
            def cph(gi):
                _, sel = selgrp(gi)
                pcv[pl.ds(gi * 16, 16)] = plsc.all_reduce_population_count(sel)

            def blk(sg, cnt2):
                cv = plsc.load_gather(pcv, [sg * 256 + lane * 16])
                csum = plsc.cumsum(cv)
                bsum = jnp.max(csum)

                @pl.when(bsum > 0)
                def _():
                    def one(i, _):
                        cnt_g = jnp.max(jnp.where(lane == i, cv, 0))
                        start = (cnt2 - cnt_g
                                 + jnp.max(jnp.where(lane == i, csum, 0)))

                        @pl.when(cnt_g > 0)
                        def _():
                            key, sel = selgrp(sg * 16 + i)
                            sel = jnp.logical_and(sel, lane + start < _CAP)
                            pos = jnp.minimum(start, _CAP - 16)
                            plsc.store_compressed(cand.at[pl.ds(pos, 16)],
                                                  key, mask=sel)
                        return 0
                    lax.fori_loop(0, 16, one, 0)
                return cnt2 + bsum
            return lax.fori_loop(0, _C // 16 // 16, blk, cnt_sofar)
        cnt1 = _dbuf_loop(g_ref, (dbuf0, dbuf1), (sem0, sem1), gbase, n,
                          relmax, nch, coll_proc, jnp.int32(0))

        ng = (cnt1 + 15) // 16

        @plsc.parallel_loop(0, ng, unroll=4)
        def bh2(gi):
            kv = cand[pl.ds(gi * 16, 16)]
            elig = (gi * 16 + lane) < cnt1
            d = jnp.where(elig,
                          lax.shift_right_arithmetic(kv, 8) & 0xFFF, 0)
            d = d | lax.shift_left(gi & 3, 12)
            cnt, last = plsc.scan_count(d)
            plsc.addupdate_scatter(hist, [d], cnt, mask=last)
        b2l, k2, _ = _scan_top(hist, 256, k1)

        @plsc.parallel_loop(0, ng, unroll=4)
        def bh3(gi):
            kv = cand[pl.ds(gi * 16, 16)]
            elig = jnp.logical_and(
                (gi * 16 + lane) < cnt1,
                (lax.shift_right_arithmetic(kv, 8) & 0xFFF) == b2l)
            d = jnp.where(elig, kv & 0xFF, 0)
            d = d | lax.shift_left(gi & 3, 12)
            cnt, last = plsc.scan_count(d)
            plsc.addupdate_scatter(hist, [d], cnt, mask=last)
        b3, _, _ = _scan_top(hist, 16, k2)

        kstar = (lax.shift_left(b1 - 2048, 20) | lax.shift_left(b2l, 8) | b3)

        def write_proc(buf, ci, ovl, lim, carry):
            @plsc.parallel_loop(0, _C // 16, unroll=4)
            def grp(gi):
                key = _to_key(buf[pl.ds(gi * 16, 16)])
                obuf[pl.ds(gi * 16, 16)] = jnp.where(
                    key >= kstar, jnp.float32(1.0), jnp.float32(0.0))

            rel = _rel_of(ci, relmax)
            obase = r * n + rel
            real = ci < nch

            @pl.when(jnp.logical_and(real, l == 0))
            def _():
                pltpu.sync_copy(obuf, out0.at[pl.ds(obase, _C)])

            @pl.when(jnp.logical_and(real, l == 1))
            def _():
                pltpu.sync_copy(obuf, out1.at[pl.ds(obase, _C)])

            @pl.when(jnp.logical_and(real, l == 2))
            def _():
                pltpu.sync_copy(obuf, out2.at[pl.ds(obase, _C)])

            @pl.when(jnp.logical_and(real, l == 3))
            def _():
                pltpu.sync_copy(obuf.at[pl.ds(0, _NS[3])],
                                out3.at[pl.ds(obase, _NS[3])])
            return carry
        _dbuf_loop(g_ref, (dbuf0, dbuf1), (sem0, sem1), gbase, n, relmax,
                   nch, write_proc, 0)

    def slot_body(slot, _):
        t = wid + slot * 32

        @pl.when(t < 4 * _TASKS)
        def _():
            run_task(t)
        return 0
    lax.fori_loop(0, 7, slot_body, 0)


@functools.partial(jax.jit, static_argnums=())
def _select_masks(gflat):
    gflat = jnp.concatenate([gflat, jnp.zeros((_C,), jnp.float32)])
    f = pl.kernel(
        _sel_body,
        mesh=plsc.VectorSubcoreMesh(core_axis_name="c", subcore_axis_name="s"),
        out_type=[jax.ShapeDtypeStruct((_TASKS * n,), jnp.float32)
                  for n in _NS],
        scratch_types=[
            pltpu.VMEM((_C,), jnp.float32),
            pltpu.VMEM((_C,), jnp.float32),
            pltpu.VMEM((_C,), jnp.float32),
            pltpu.VMEM((4 * 4096,), jnp.int32),
            pltpu.VMEM((_CAP,), jnp.int32),
            pltpu.VMEM((_C,), jnp.int32),
            pltpu.SemaphoreType.DMA,
            pltpu.SemaphoreType.DMA,
        ],
        compiler_params=pltpu.CompilerParams(needs_layout_passes=False),
    )
    return f(gflat)


def kernel(x, embedding_input, W1, b1, W2, b2):
    g = _compute_g(embedding_input, W1, b1, W2, b2)
    m0, m1, m2, m3 = _select_masks(g.reshape(-1))
    outs = []
    for l, m in enumerate((m0, m1, m2, m3)):
        in_d, out_d = _DIMS[l], _DIMS[l + 1]
        outs.append(m.reshape(_TASKS, out_d, in_d))
    return tuple(outs)
